# baseline (speedup 1.0000x reference)
import functools

import jax
import jax.numpy as jnp
from jax import lax
from jax.experimental import pallas as pl
from jax.experimental.pallas import tpu as pltpu
from jax.experimental.pallas import tpu_sc as plsc

D = 16
NW = 32
CHUNK = 1024
NSUB = 8
SUB = CHUNK // NSUB
NCHUNK = 50
ROWS_W = CHUNK * NCHUNK
NSEG = 100 * 2 * 16
EPS = 1e-7


def _build_sc_call():
    mesh = plsc.VectorSubcoreMesh(core_axis_name="c", subcore_axis_name="s")

    @functools.partial(
        pl.kernel,
        out_type=jax.ShapeDtypeStruct((NSEG, 8, 8, 128), jnp.float32),
        mesh=mesh,
        compiler_params=pltpu.CompilerParams(
            needs_layout_passes=False, use_tc_tiling_on_sc=False),
        scratch_types=[
            pltpu.VMEM((NCHUNK, NSUB, SUB), jnp.int32),
            pltpu.VMEM((CHUNK, D), jnp.float32),
            pltpu.VMEM((CHUNK, D), jnp.float32),
            pltpu.VMEM((2, 8, 8, 128), jnp.float32),
            pltpu.VMEM((2, 8, 8, 128), jnp.float32),
            pltpu.SemaphoreType.DMA,
            pltpu.SemaphoreType.DMA,
            pltpu.SemaphoreType.DMA,
            pltpu.SemaphoreType.DMA,
        ],
    )
    def sc_fn(idx_hbm, table_hbm, out_hbm,
              idx_v, rows0, rows1, outt0, outt1, g0, g1, o0, o1):
        wid = lax.axis_index("s") * 2 + lax.axis_index("c")
        rows = (rows0, rows1)
        outt = (outt0, outt1)
        gsem = (g0, g1)
        osem = (o0, o1)

        pltpu.sync_copy(idx_hbm.at[wid], idx_v)

        iota = lax.iota(jnp.int32, 16)
        iota16 = iota * 16
        zeros = jnp.zeros((16,), jnp.int32)
        magic = jnp.full((16,), 0x5F3759DF, jnp.int32)
        c15 = jnp.full((16,), 1.5, jnp.float32)
        c05 = jnp.full((16,), 0.5, jnp.float32)
        one = jnp.full((16,), 1.0, jnp.float32)
        eps = jnp.full((16,), EPS, jnp.float32)

        def fire_gathers(c, b):
            for j in range(NSUB):
                pltpu.async_copy(
                    table_hbm.at[idx_v.at[c, j]],
                    rows[b].at[pl.ds(j * SUB, SUB)],
                    gsem[b])

        def drain_gathers(b):
            pltpu.make_async_copy(
                table_hbm.at[pl.ds(0, CHUNK)], rows[b], gsem[b]).wait()

        def compute(b):
            rref = rows[b]
            oref = outt[b]

            def group(g2):
                bb = g2 >> 3
                bl0 = (g2 & 7) * 16
                fbase = g2 * 256 + iota16
                vs = []
                s = None
                for d in range(D):
                    v = plsc.load_gather(rref, [zeros, fbase + d])
                    vs.append(v)
                    s = v * v if s is None else s + v * v
                bits = plsc.bitcast(s, jnp.int32)
                y = plsc.bitcast(magic - (bits >> 1), jnp.float32)
                for _ in range(3):
                    y = y * (c15 - c05 * s * y * y)
                scale = y - eps * y * y
                scale = jnp.where(s > one, scale, one)
                for d in range(D):
                    oref[d // 8, bb, d % 8, pl.ds(bl0, 16)] = vs[d] * scale

            def body(g4, carry):
                group(g4 * 2)
                group(g4 * 2 + 1)
                return carry

            lax.fori_loop(0, CHUNK // 32, body, None)

        def fire_out(c, b):
            g = wid * NCHUNK + c
            f = g >> 4
            bb8 = g & 15
            for dg in range(2):
                pltpu.async_copy(
                    outt[b].at[dg],
                    out_hbm.at[f * 32 + dg * 16 + bb8],
                    osem[b])

        def drain_out(b):
            pltpu.make_async_copy(
                out_hbm.at[pl.ds(0, 2)], outt[b], osem[b]).wait()

        def handle(c, b, first, prefetch):
            if prefetch:
                fire_gathers(c + 1, 1 - b)
            drain_gathers(b)
            if not first:
                drain_out(b)
            compute(b)
            fire_out(c, b)

        fire_gathers(0, 0)
        handle(0, 0, True, True)
        handle(1, 1, True, True)

        def loop_body(it, carry):
            handle(2 * it, 0, False, True)
            handle(2 * it + 1, 1, False, True)
            return carry

        lax.fori_loop(1, NCHUNK // 2 - 1, loop_body, None)

        handle(NCHUNK - 2, 0, False, True)
        handle(NCHUNK - 1, 1, False, False)
        drain_out(0)
        drain_out(1)

    return sc_fn


_sc_call = _build_sc_call()


NBLK = 7812
BLK_W = NBLK // NW


def _build_tr_call():
    mesh = plsc.VectorSubcoreMesh(core_axis_name="c", subcore_axis_name="s")

    @functools.partial(
        pl.kernel,
        out_type=jax.ShapeDtypeStruct((15625, 8, 128), jnp.float32),
        mesh=mesh,
        compiler_params=pltpu.CompilerParams(
            needs_layout_passes=False, use_tc_tiling_on_sc=True),
        scratch_types=[
            pltpu.VMEM((2, 2, 8, 128), jnp.float32),
            pltpu.VMEM((2, 2, 8, 128), jnp.float32),
            pltpu.SemaphoreType.DMA,
            pltpu.SemaphoreType.DMA,
            pltpu.SemaphoreType.DMA,
            pltpu.SemaphoreType.DMA,
        ],
    )
    def tr_fn(tab_hbm, tail_hbm, out_hbm, inb, outb, i0, i1, o0, o1):
        wid = lax.axis_index("s") * 2 + lax.axis_index("c")
        base = wid * BLK_W
        isem = (i0, i1)
        osem = (o0, o1)

        iota = lax.iota(jnp.int32, 16)
        iota128 = iota * 128
        zeros = jnp.zeros((16,), jnp.int32)

        def fire_in(k, b):
            pltpu.async_copy(
                tab_hbm.at[:, :, pl.ds((base + k) * 128, 128)],
                inb.at[b], isem[b])

        def drain_in(b):
            pltpu.make_async_copy(
                tab_hbm.at[:, :, pl.ds(0, 128)], inb.at[b], isem[b]).wait()

        def transpose_block(b):
            src = inb.at[b]
            dst = outb.at[b]
            for i in range(16):
                for rr in range(8):
                    v = plsc.load_gather(
                        src, [zeros, zeros, iota128 + (i * 8 + rr)])
                    dst[i // 8, i % 8, pl.ds(rr * 16, 16)] = v

        def fire_out(k, b):
            pltpu.async_copy(
                outb.at[b], out_hbm.at[pl.ds((base + k) * 2, 2)], osem[b])

        def drain_out(b):
            pltpu.make_async_copy(
                outb.at[b], out_hbm.at[pl.ds(0, 2)], osem[b]).wait()

        def handle(k, b, first, prefetch):
            if prefetch:
                fire_in(k + 1, 1 - b)
            drain_in(b)
            if not first:
                drain_out(b)
            transpose_block(b)
            fire_out(k, b)

        fire_in(0, 0)
        handle(0, 0, True, True)
        handle(1, 1, True, True)

        def loop_body(it, carry):
            handle(2 * it, 0, False, True)
            handle(2 * it + 1, 1, False, True)
            return carry

        lax.fori_loop(1, BLK_W // 2 - 1, loop_body, None)

        handle(BLK_W - 2, 0, False, True)
        handle(BLK_W - 1, 1, False, False)
        drain_out(0)
        drain_out(1)

        for e in range(4):
            @pl.when(wid == e)
            def _():
                blk = NBLK - 4 + e
                pltpu.sync_copy(
                    tab_hbm.at[:, :, pl.ds(blk * 128, 128)], inb.at[0])
                transpose_block(0)
                pltpu.sync_copy(
                    outb.at[0], out_hbm.at[pl.ds(blk * 2, 2)])

        @pl.when(wid == 4)
        def _():
            pltpu.sync_copy(tail_hbm, outb.at[0, pl.ds(0, 1)])
            pltpu.sync_copy(
                outb.at[0, pl.ds(0, 1)], out_hbm.at[pl.ds(NBLK * 2, 1)])

    return tr_fn


_tr_call = _build_tr_call()


def kernel(idx, table):
    B, F = idx.shape
    flat = idx.astype(jnp.int32).T.reshape(NW, NCHUNK, NSUB, SUB)
    tail = table[NBLK * 128:, :].reshape(1, 8, 128)
    table_lin = _tr_call(
        table.T.reshape(2, 8, 1000000), tail).reshape(1000000, D)
    out = _sc_call(flat, table_lin)
    o = out.reshape(F, 2, 16, 8, 8, 128)
    o = o.transpose(2, 3, 5, 0, 1, 4)
    return o.reshape(B, F, D)


# speedup vs baseline: 1.0216x; 1.0216x over previous
import functools

import jax
import jax.numpy as jnp
from jax import lax
from jax.experimental import pallas as pl
from jax.experimental.pallas import tpu as pltpu
from jax.experimental.pallas import tpu_sc as plsc

D = 16
NW = 32
CHUNK = 1024
NSUB = 8
SUB = CHUNK // NSUB
NCHUNK = 50
ROWS_W = CHUNK * NCHUNK
NSEG = 100 * 2 * 16
EPS = 1e-7


def _build_sc_call():
    mesh = plsc.VectorSubcoreMesh(core_axis_name="c", subcore_axis_name="s")

    @functools.partial(
        pl.kernel,
        out_type=jax.ShapeDtypeStruct((NSEG, 8, 8, 128), jnp.float32),
        mesh=mesh,
        compiler_params=pltpu.CompilerParams(
            needs_layout_passes=False, use_tc_tiling_on_sc=False),
        scratch_types=[
            pltpu.VMEM((NCHUNK, NSUB, SUB), jnp.int32),
            pltpu.VMEM((CHUNK, D), jnp.float32),
            pltpu.VMEM((CHUNK, D), jnp.float32),
            pltpu.VMEM((2, 8, 8, 128), jnp.float32),
            pltpu.VMEM((2, 8, 8, 128), jnp.float32),
            pltpu.SemaphoreType.DMA,
            pltpu.SemaphoreType.DMA,
            pltpu.SemaphoreType.DMA,
            pltpu.SemaphoreType.DMA,
        ],
    )
    def sc_fn(idx_hbm, table_hbm, out_hbm,
              idx_v, rows0, rows1, outt0, outt1, g0, g1, o0, o1):
        wid = lax.axis_index("s") * 2 + lax.axis_index("c")
        rows = (rows0, rows1)
        outt = (outt0, outt1)
        gsem = (g0, g1)
        osem = (o0, o1)

        pltpu.sync_copy(idx_hbm.at[wid], idx_v)

        iota = lax.iota(jnp.int32, 16)
        iota16 = iota * 16
        zeros = jnp.zeros((16,), jnp.int32)
        magic = jnp.full((16,), 0x5F3759DF, jnp.int32)
        c15 = jnp.full((16,), 1.5, jnp.float32)
        c05 = jnp.full((16,), 0.5, jnp.float32)
        one = jnp.full((16,), 1.0, jnp.float32)
        eps = jnp.full((16,), EPS, jnp.float32)

        def fire_gathers(c, b):
            for j in range(NSUB):
                pltpu.async_copy(
                    table_hbm.at[idx_v.at[c, j]],
                    rows[b].at[pl.ds(j * SUB, SUB)],
                    gsem[b])

        def drain_gathers(b):
            pltpu.make_async_copy(
                table_hbm.at[pl.ds(0, CHUNK)], rows[b], gsem[b]).wait()

        def compute(b):
            rref = rows[b]
            oref = outt[b]

            def group(g2):
                bb = g2 >> 3
                bl0 = (g2 & 7) * 16
                fbase = g2 * 256 + iota16
                vs = []
                s = None
                for d in range(D):
                    v = plsc.load_gather(rref, [zeros, fbase + d])
                    vs.append(v)
                    s = v * v if s is None else s + v * v
                bits = plsc.bitcast(s, jnp.int32)
                y = plsc.bitcast(magic - (bits >> 1), jnp.float32)
                for _ in range(3):
                    y = y * (c15 - c05 * s * y * y)
                scale = y - eps * y * y
                scale = jnp.where(s > one, scale, one)
                for d in range(D):
                    oref[d // 8, bb, d % 8, pl.ds(bl0, 16)] = vs[d] * scale

            def body(g4, carry):
                group(g4 * 2)
                group(g4 * 2 + 1)
                return carry

            lax.fori_loop(0, CHUNK // 32, body, None)

        def fire_out(c, b):
            g = wid * NCHUNK + c
            f = g >> 4
            bb8 = g & 15
            for dg in range(2):
                pltpu.async_copy(
                    outt[b].at[dg],
                    out_hbm.at[f * 32 + dg * 16 + bb8],
                    osem[b])

        def drain_out(b):
            pltpu.make_async_copy(
                out_hbm.at[pl.ds(0, 2)], outt[b], osem[b]).wait()

        def handle(c, b, first, prefetch):
            if prefetch:
                fire_gathers(c + 1, 1 - b)
            drain_gathers(b)
            if not first:
                drain_out(b)
            compute(b)
            fire_out(c, b)

        fire_gathers(0, 0)
        handle(0, 0, True, True)
        handle(1, 1, True, True)

        def loop_body(it, carry):
            handle(2 * it, 0, False, True)
            handle(2 * it + 1, 1, False, True)
            return carry

        lax.fori_loop(1, NCHUNK // 2 - 1, loop_body, None)

        handle(NCHUNK - 2, 0, False, True)
        handle(NCHUNK - 1, 1, False, False)
        drain_out(0)
        drain_out(1)

    return sc_fn


_sc_call = _build_sc_call()


NBLK = 7812
BLK_W = NBLK // NW
NBUF = 4


def _build_tr_call():
    mesh = plsc.VectorSubcoreMesh(core_axis_name="c", subcore_axis_name="s")

    @functools.partial(
        pl.kernel,
        out_type=jax.ShapeDtypeStruct((15625, 8, 128), jnp.float32),
        mesh=mesh,
        compiler_params=pltpu.CompilerParams(
            needs_layout_passes=False, use_tc_tiling_on_sc=True),
        scratch_types=[
            pltpu.VMEM((NBUF, 2, 8, 128), jnp.float32),
            pltpu.VMEM((NBUF, 2, 8, 128), jnp.float32),
        ] + [pltpu.SemaphoreType.DMA] * (2 * NBUF),
    )
    def tr_fn(tab_hbm, tail_hbm, out_hbm, inb, outb, *sems):
        wid = lax.axis_index("s") * 2 + lax.axis_index("c")
        base = wid * BLK_W
        isem = sems[:NBUF]
        osem = sems[NBUF:]

        iota = lax.iota(jnp.int32, 16)
        iota128 = iota * 128
        zeros = jnp.zeros((16,), jnp.int32)

        def fire_in(k, b):
            pltpu.async_copy(
                tab_hbm.at[:, :, pl.ds((base + k) * 128, 128)],
                inb.at[b], isem[b])

        def drain_in(b):
            pltpu.make_async_copy(
                tab_hbm.at[:, :, pl.ds(0, 128)], inb.at[b], isem[b]).wait()

        def transpose_block(b):
            src = inb.at[b]
            dst = outb.at[b]

            def body(i, carry):
                ibase = iota128 + i * 8
                for rr in range(8):
                    v = plsc.load_gather(src, [zeros, zeros, ibase + rr])
                    dst[i >> 3, i & 7, pl.ds(rr * 16, 16)] = v
                return carry

            lax.fori_loop(0, 16, body, None)

        def fire_out(k, b):
            pltpu.async_copy(
                outb.at[b], out_hbm.at[pl.ds((base + k) * 2, 2)], osem[b])

        def drain_out(b):
            pltpu.make_async_copy(
                outb.at[b], out_hbm.at[pl.ds(0, 2)], osem[b]).wait()

        def handle(k, b, first, prefetch):
            if prefetch:
                fire_in(k + NBUF - 1, (b + NBUF - 1) % NBUF)
            drain_in(b)
            if not first:
                drain_out(b)
            transpose_block(b)
            fire_out(k, b)

        for b in range(NBUF - 1):
            fire_in(b, b)
        for k in range(NBUF):
            handle(k, k, True, True)

        def loop_body(it, carry):
            for b in range(NBUF):
                handle(it * NBUF + b, b, False, True)
            return carry

        lax.fori_loop(1, BLK_W // NBUF - 1, loop_body, None)

        for k in range(BLK_W - NBUF, BLK_W):
            handle(k, k % NBUF, False, k + NBUF - 1 < BLK_W)
        for b in range(NBUF):
            drain_out(b)

        for e in range(4):
            @pl.when(wid == e)
            def _():
                blk = NBLK - 4 + e
                pltpu.sync_copy(
                    tab_hbm.at[:, :, pl.ds(blk * 128, 128)], inb.at[0])
                transpose_block(0)
                pltpu.sync_copy(
                    outb.at[0], out_hbm.at[pl.ds(blk * 2, 2)])

        @pl.when(wid == 4)
        def _():
            pltpu.sync_copy(tail_hbm, outb.at[0, pl.ds(0, 1)])
            pltpu.sync_copy(
                outb.at[0, pl.ds(0, 1)], out_hbm.at[pl.ds(NBLK * 2, 1)])

    return tr_fn


_tr_call = _build_tr_call()


def kernel(idx, table):
    B, F = idx.shape
    flat = idx.astype(jnp.int32).T.reshape(NW, NCHUNK, NSUB, SUB)
    tail = table[NBLK * 128:, :].reshape(1, 8, 128)
    table_lin = _tr_call(
        table.T.reshape(2, 8, 1000000), tail).reshape(1000000, D)
    out = _sc_call(flat, table_lin)
    o = out.reshape(F, 2, 16, 8, 8, 128)
    o = o.transpose(2, 3, 5, 0, 1, 4)
    return o.reshape(B, F, D)


# speedup vs baseline: 1.7385x; 1.7018x over previous
import functools

import jax
import jax.numpy as jnp
from jax import lax
from jax.experimental import pallas as pl
from jax.experimental.pallas import tpu as pltpu
from jax.experimental.pallas import tpu_sc as plsc

D = 16
NW = 32
CHUNK = 1024
NSUB = 8
SUB = CHUNK // NSUB
NCHUNK = 50
ROWS_W = CHUNK * NCHUNK
NSEG = 100 * 2 * 16
EPS = 1e-7


def _build_sc_call():
    mesh = plsc.VectorSubcoreMesh(core_axis_name="c", subcore_axis_name="s")

    @functools.partial(
        pl.kernel,
        out_type=jax.ShapeDtypeStruct((NSEG, 8, 8, 128), jnp.float32),
        mesh=mesh,
        compiler_params=pltpu.CompilerParams(
            needs_layout_passes=False, use_tc_tiling_on_sc=False),
        scratch_types=[
            pltpu.VMEM((NCHUNK, NSUB, SUB), jnp.int32),
            pltpu.VMEM((CHUNK, D), jnp.float32),
            pltpu.VMEM((CHUNK, D), jnp.float32),
            pltpu.VMEM((2, 8, 8, 128), jnp.float32),
            pltpu.VMEM((2, 8, 8, 128), jnp.float32),
            pltpu.SemaphoreType.DMA,
            pltpu.SemaphoreType.DMA,
            pltpu.SemaphoreType.DMA,
            pltpu.SemaphoreType.DMA,
        ],
    )
    def sc_fn(idx_hbm, table_hbm, out_hbm,
              idx_v, rows0, rows1, outt0, outt1, g0, g1, o0, o1):
        wid = lax.axis_index("s") * 2 + lax.axis_index("c")
        rows = (rows0, rows1)
        outt = (outt0, outt1)
        gsem = (g0, g1)
        osem = (o0, o1)

        pltpu.sync_copy(idx_hbm.at[wid], idx_v)

        iota = lax.iota(jnp.int32, 16)
        iota16 = iota * 16
        zeros = jnp.zeros((16,), jnp.int32)
        magic = jnp.full((16,), 0x5F3759DF, jnp.int32)
        c15 = jnp.full((16,), 1.5, jnp.float32)
        c05 = jnp.full((16,), 0.5, jnp.float32)
        one = jnp.full((16,), 1.0, jnp.float32)
        eps = jnp.full((16,), EPS, jnp.float32)

        def fire_gathers(c, b):
            for j in range(NSUB):
                pltpu.async_copy(
                    table_hbm.at[idx_v.at[c, j]],
                    rows[b].at[pl.ds(j * SUB, SUB)],
                    gsem[b])

        def drain_gathers(b):
            pltpu.make_async_copy(
                table_hbm.at[pl.ds(0, CHUNK)], rows[b], gsem[b]).wait()

        def compute(b):
            rref = rows[b]
            oref = outt[b]

            def group(g2):
                bb = g2 >> 3
                bl0 = (g2 & 7) * 16
                fbase = g2 * 256 + iota16
                vs = []
                s = None
                for d in range(D):
                    v = plsc.load_gather(rref, [zeros, fbase + d])
                    vs.append(v)
                    s = v * v if s is None else s + v * v
                bits = plsc.bitcast(s, jnp.int32)
                y = plsc.bitcast(magic - (bits >> 1), jnp.float32)
                for _ in range(3):
                    y = y * (c15 - c05 * s * y * y)
                scale = y - eps * y * y
                scale = jnp.where(s > one, scale, one)
                for d in range(D):
                    oref[d // 8, bb, d % 8, pl.ds(bl0, 16)] = vs[d] * scale

            def body(g4, carry):
                group(g4 * 2)
                group(g4 * 2 + 1)
                return carry

            lax.fori_loop(0, CHUNK // 32, body, None)

        def fire_out(c, b):
            g = wid * NCHUNK + c
            f = g >> 4
            bb8 = g & 15
            for dg in range(2):
                pltpu.async_copy(
                    outt[b].at[dg],
                    out_hbm.at[f * 32 + dg * 16 + bb8],
                    osem[b])

        def drain_out(b):
            pltpu.make_async_copy(
                out_hbm.at[pl.ds(0, 2)], outt[b], osem[b]).wait()

        def handle(c, b, first, prefetch):
            if prefetch:
                fire_gathers(c + 1, 1 - b)
            drain_gathers(b)
            if not first:
                drain_out(b)
            compute(b)
            fire_out(c, b)

        fire_gathers(0, 0)
        handle(0, 0, True, True)
        handle(1, 1, True, True)

        def loop_body(it, carry):
            handle(2 * it, 0, False, True)
            handle(2 * it + 1, 1, False, True)
            return carry

        lax.fori_loop(1, NCHUNK // 2 - 1, loop_body, None)

        handle(NCHUNK - 2, 0, False, True)
        handle(NCHUNK - 1, 1, False, False)
        drain_out(0)
        drain_out(1)

    return sc_fn


_sc_call = _build_sc_call()


NBLK = 7812
BLK_W = NBLK // NW
NBUF = 4


def _build_tr_call():
    mesh = plsc.VectorSubcoreMesh(core_axis_name="c", subcore_axis_name="s")

    @functools.partial(
        pl.kernel,
        out_type=jax.ShapeDtypeStruct((15625, 8, 128), jnp.float32),
        mesh=mesh,
        compiler_params=pltpu.CompilerParams(
            needs_layout_passes=False, use_tc_tiling_on_sc=True),
        scratch_types=[
            pltpu.VMEM((NBUF, 2, 8, 128), jnp.float32),
            pltpu.VMEM((NBUF, 2, 8, 128), jnp.float32),
        ] + [pltpu.SemaphoreType.DMA] * (2 * NBUF),
    )
    def tr_fn(tab_hbm, tail_hbm, out_hbm, inb, outb, *sems):
        wid = lax.axis_index("s") * 2 + lax.axis_index("c")
        base = wid * BLK_W
        isem = sems[:NBUF]
        osem = sems[NBUF:]

        iota = lax.iota(jnp.int32, 16)
        iota7 = iota & 7
        zeros = jnp.zeros((16,), jnp.int32)

        def fire_in(k, b):
            pltpu.async_copy(
                tab_hbm.at[:, :, pl.ds((base + k) * 128, 128)],
                inb.at[b], isem[b])

        def drain_in(b):
            pltpu.make_async_copy(
                tab_hbm.at[:, :, pl.ds(0, 128)], inb.at[b], isem[b]).wait()

        def transpose_block(b):
            src = inb.at[b]
            dst = outb.at[b]

            def body(mm, carry):
                m0 = mm * 16
                ivec = (m0 + iota) >> 3
                fbase = (ivec >> 3) * 1024 + (ivec & 7) * 128 + iota7 * 16
                for g in range(2):
                    for r in range(8):
                        v = src[g, r, pl.ds(m0, 16)]
                        plsc.store_scatter(
                            dst, [zeros, zeros, fbase + (g * 8 + r)], v)
                return carry

            lax.fori_loop(0, 8, body, None)

        def fire_out(k, b):
            pltpu.async_copy(
                outb.at[b], out_hbm.at[pl.ds((base + k) * 2, 2)], osem[b])

        def drain_out(b):
            pltpu.make_async_copy(
                outb.at[b], out_hbm.at[pl.ds(0, 2)], osem[b]).wait()

        def handle(k, b, first, prefetch):
            if prefetch:
                fire_in(k + NBUF - 1, (b + NBUF - 1) % NBUF)
            drain_in(b)
            if not first:
                drain_out(b)
            transpose_block(b)
            fire_out(k, b)

        for b in range(NBUF - 1):
            fire_in(b, b)
        for k in range(NBUF):
            handle(k, k, True, True)

        def loop_body(it, carry):
            for b in range(NBUF):
                handle(it * NBUF + b, b, False, True)
            return carry

        lax.fori_loop(1, BLK_W // NBUF - 1, loop_body, None)

        for k in range(BLK_W - NBUF, BLK_W):
            handle(k, k % NBUF, False, k + NBUF - 1 < BLK_W)
        for b in range(NBUF):
            drain_out(b)

        for e in range(4):
            @pl.when(wid == e)
            def _():
                blk = NBLK - 4 + e
                pltpu.sync_copy(
                    tab_hbm.at[:, :, pl.ds(blk * 128, 128)],
                    inb.at[0, :, :, pl.ds(0, 128)])
                transpose_block(0)
                pltpu.sync_copy(
                    outb.at[0], out_hbm.at[pl.ds(blk * 2, 2)])

        @pl.when(wid == 4)
        def _():
            pltpu.sync_copy(tail_hbm, outb.at[0, pl.ds(0, 1)])
            pltpu.sync_copy(
                outb.at[0, pl.ds(0, 1)], out_hbm.at[pl.ds(NBLK * 2, 1)])

    return tr_fn


_tr_call = _build_tr_call()


def kernel(idx, table):
    B, F = idx.shape
    flat = idx.astype(jnp.int32).T.reshape(NW, NCHUNK, NSUB, SUB)
    tail = table[NBLK * 128:, :].reshape(1, 8, 128)
    table_lin = _tr_call(
        table.T.reshape(2, 8, 1000000), tail).reshape(1000000, D)
    out = _sc_call(flat, table_lin)
    o = out.reshape(F, 2, 16, 8, 8, 128)
    o = o.transpose(2, 3, 5, 0, 1, 4)
    return o.reshape(B, F, D)


# speedup vs baseline: 1.7648x; 1.0151x over previous
import functools

import jax
import jax.numpy as jnp
from jax import lax
from jax.experimental import pallas as pl
from jax.experimental.pallas import tpu as pltpu
from jax.experimental.pallas import tpu_sc as plsc

D = 16
NW = 32
CHUNK = 1024
NSUB = 8
SUB = CHUNK // NSUB
NCHUNK = 50
ROWS_W = CHUNK * NCHUNK
NSEG = 100 * 2 * 16
EPS = 1e-7


def _build_sc_call():
    mesh = plsc.VectorSubcoreMesh(core_axis_name="c", subcore_axis_name="s")

    @functools.partial(
        pl.kernel,
        out_type=jax.ShapeDtypeStruct((NSEG, 8, 8, 128), jnp.float32),
        mesh=mesh,
        compiler_params=pltpu.CompilerParams(
            needs_layout_passes=False, use_tc_tiling_on_sc=False),
        scratch_types=[
            pltpu.VMEM((NCHUNK, NSUB, SUB), jnp.int32),
            pltpu.VMEM((CHUNK, D), jnp.float32),
            pltpu.VMEM((CHUNK, D), jnp.float32),
            pltpu.VMEM((2, 8, 8, 128), jnp.float32),
            pltpu.VMEM((2, 8, 8, 128), jnp.float32),
            pltpu.SemaphoreType.DMA,
            pltpu.SemaphoreType.DMA,
            pltpu.SemaphoreType.DMA,
            pltpu.SemaphoreType.DMA,
        ],
    )
    def sc_fn(idx_hbm, table_hbm, out_hbm,
              idx_v, rows0, rows1, outt0, outt1, g0, g1, o0, o1):
        wid = lax.axis_index("s") * 2 + lax.axis_index("c")
        rows = (rows0, rows1)
        outt = (outt0, outt1)
        gsem = (g0, g1)
        osem = (o0, o1)

        pltpu.sync_copy(idx_hbm.at[wid], idx_v)

        iota = lax.iota(jnp.int32, 16)
        iota16 = iota * 16
        zeros = jnp.zeros((16,), jnp.int32)
        magic = jnp.full((16,), 0x5F3759DF, jnp.int32)
        c15 = jnp.full((16,), 1.5, jnp.float32)
        c05 = jnp.full((16,), 0.5, jnp.float32)
        one = jnp.full((16,), 1.0, jnp.float32)
        eps = jnp.full((16,), EPS, jnp.float32)

        def fire_gathers(c, b):
            for j in range(NSUB):
                pltpu.async_copy(
                    table_hbm.at[idx_v.at[c, j]],
                    rows[b].at[pl.ds(j * SUB, SUB)],
                    gsem[b])

        def drain_gathers(b):
            pltpu.make_async_copy(
                table_hbm.at[pl.ds(0, CHUNK)], rows[b], gsem[b]).wait()

        def compute(b):
            rref = rows[b]
            oref = outt[b]

            def group(g2):
                bb = g2 >> 3
                bl0 = (g2 & 7) * 16
                fbase = g2 * 256 + iota16
                vs = []
                s = None
                for d in range(D):
                    v = plsc.load_gather(rref, [zeros, fbase + d])
                    vs.append(v)
                    s = v * v if s is None else s + v * v
                bits = plsc.bitcast(s, jnp.int32)
                y = plsc.bitcast(magic - (bits >> 1), jnp.float32)
                for _ in range(2):
                    y = y * (c15 - c05 * s * y * y)
                scale = y - eps * y * y
                scale = jnp.where(s > one, scale, one)
                for d in range(D):
                    oref[d // 8, bb, d % 8, pl.ds(bl0, 16)] = vs[d] * scale

            def body(g4, carry):
                for u in range(4):
                    group(g4 * 4 + u)
                return carry

            lax.fori_loop(0, CHUNK // 64, body, None)

        def fire_out(c, b):
            g = wid * NCHUNK + c
            f = g >> 4
            bb8 = g & 15
            for dg in range(2):
                pltpu.async_copy(
                    outt[b].at[dg],
                    out_hbm.at[f * 32 + dg * 16 + bb8],
                    osem[b])

        def drain_out(b):
            pltpu.make_async_copy(
                out_hbm.at[pl.ds(0, 2)], outt[b], osem[b]).wait()

        def handle(c, b, first, prefetch):
            if prefetch:
                fire_gathers(c + 1, 1 - b)
            drain_gathers(b)
            if not first:
                drain_out(b)
            compute(b)
            fire_out(c, b)

        fire_gathers(0, 0)
        handle(0, 0, True, True)
        handle(1, 1, True, True)

        def loop_body(it, carry):
            handle(2 * it, 0, False, True)
            handle(2 * it + 1, 1, False, True)
            return carry

        lax.fori_loop(1, NCHUNK // 2 - 1, loop_body, None)

        handle(NCHUNK - 2, 0, False, True)
        handle(NCHUNK - 1, 1, False, False)
        drain_out(0)
        drain_out(1)

    return sc_fn


_sc_call = _build_sc_call()


NBLK = 7812
BLK_W = NBLK // NW
NBUF = 4


def _build_tr_call():
    mesh = plsc.VectorSubcoreMesh(core_axis_name="c", subcore_axis_name="s")

    @functools.partial(
        pl.kernel,
        out_type=jax.ShapeDtypeStruct((15625, 8, 128), jnp.float32),
        mesh=mesh,
        compiler_params=pltpu.CompilerParams(
            needs_layout_passes=False, use_tc_tiling_on_sc=True),
        scratch_types=[
            pltpu.VMEM((NBUF, 2, 8, 128), jnp.float32),
            pltpu.VMEM((NBUF, 2, 8, 128), jnp.float32),
        ] + [pltpu.SemaphoreType.DMA] * (2 * NBUF),
    )
    def tr_fn(tab_hbm, tail_hbm, out_hbm, inb, outb, *sems):
        wid = lax.axis_index("s") * 2 + lax.axis_index("c")
        base = wid * BLK_W
        isem = sems[:NBUF]
        osem = sems[NBUF:]

        iota = lax.iota(jnp.int32, 16)
        iota7 = iota & 7
        zeros = jnp.zeros((16,), jnp.int32)

        def fire_in(k, b):
            pltpu.async_copy(
                tab_hbm.at[:, :, pl.ds((base + k) * 128, 128)],
                inb.at[b], isem[b])

        def drain_in(b):
            pltpu.make_async_copy(
                tab_hbm.at[:, :, pl.ds(0, 128)], inb.at[b], isem[b]).wait()

        def transpose_block(b):
            src = inb.at[b]
            dst = outb.at[b]

            def body(mm, carry):
                m0 = mm * 16
                ivec = (m0 + iota) >> 3
                fbase = (ivec >> 3) * 1024 + (ivec & 7) * 128 + iota7 * 16
                for g in range(2):
                    for r in range(8):
                        v = src[g, r, pl.ds(m0, 16)]
                        plsc.store_scatter(
                            dst, [zeros, zeros, fbase + (g * 8 + r)], v)
                return carry

            lax.fori_loop(0, 8, body, None)

        def fire_out(k, b):
            pltpu.async_copy(
                outb.at[b], out_hbm.at[pl.ds((base + k) * 2, 2)], osem[b])

        def drain_out(b):
            pltpu.make_async_copy(
                outb.at[b], out_hbm.at[pl.ds(0, 2)], osem[b]).wait()

        def handle(k, b, first, prefetch):
            if prefetch:
                fire_in(k + NBUF - 1, (b + NBUF - 1) % NBUF)
            drain_in(b)
            if not first:
                drain_out(b)
            transpose_block(b)
            fire_out(k, b)

        for b in range(NBUF - 1):
            fire_in(b, b)
        for k in range(NBUF):
            handle(k, k, True, True)

        def loop_body(it, carry):
            for b in range(NBUF):
                handle(it * NBUF + b, b, False, True)
            return carry

        lax.fori_loop(1, BLK_W // NBUF - 1, loop_body, None)

        for k in range(BLK_W - NBUF, BLK_W):
            handle(k, k % NBUF, False, k + NBUF - 1 < BLK_W)
        for b in range(NBUF):
            drain_out(b)

        for e in range(4):
            @pl.when(wid == e)
            def _():
                blk = NBLK - 4 + e
                pltpu.sync_copy(
                    tab_hbm.at[:, :, pl.ds(blk * 128, 128)],
                    inb.at[0, :, :, pl.ds(0, 128)])
                transpose_block(0)
                pltpu.sync_copy(
                    outb.at[0], out_hbm.at[pl.ds(blk * 2, 2)])

        @pl.when(wid == 4)
        def _():
            pltpu.sync_copy(tail_hbm, outb.at[0, pl.ds(0, 1)])
            pltpu.sync_copy(
                outb.at[0, pl.ds(0, 1)], out_hbm.at[pl.ds(NBLK * 2, 1)])

    return tr_fn


_tr_call = _build_tr_call()


def kernel(idx, table):
    B, F = idx.shape
    flat = idx.astype(jnp.int32).T.reshape(NW, NCHUNK, NSUB, SUB)
    tail = table[NBLK * 128:, :].reshape(1, 8, 128)
    table_lin = _tr_call(
        table.T.reshape(2, 8, 1000000), tail).reshape(1000000, D)
    out = _sc_call(flat, table_lin)
    o = out.reshape(F, 2, 16, 8, 8, 128)
    o = o.transpose(2, 3, 5, 0, 1, 4)
    return o.reshape(B, F, D)
